# baseline (speedup 1.0000x reference)
import functools

import jax
import jax.numpy as jnp
from jax import lax
from jax.experimental import pallas as pl
from jax.experimental.pallas import tpu as pltpu
from jax.experimental.pallas import tpu_sc as plsc

_NC = 2
_NS = 16
_NW = _NC * _NS
_L = 16
_K = 32
_B = 16
_MGR = 512
_RAT = 24

_O_WADJ = 0
_O_EM = _O_WADJ + _B * _K
_O_ER = _O_EM + _MGR
_O_KNOTS = 1048
_O_WBASE = _O_KNOTS + _K
_O_WWAL = _O_WBASE + _K
_O_WDIV = _O_WWAL + _K
_O_BADJ = _O_WDIV + _K
_O_SCAL = _O_BADJ + _B
_WTAB = 1208


def _cumsum2(lo, hi):
    clo = jnp.cumsum(lo)
    return clo, jnp.cumsum(hi) + jnp.sum(lo)


@functools.lru_cache(maxsize=None)
def _sc_call(n):
    rpw = n // _NW
    mesh = plsc.VectorSubcoreMesh(core_axis_name="c", subcore_axis_name="s",
                                  num_cores=_NC)

    @functools.partial(
        pl.kernel,
        mesh=mesh,
        compiler_params=pltpu.CompilerParams(needs_layout_passes=False),
        out_type=jax.ShapeDtypeStruct((n,), jnp.float32),
        scratch_types=[
            pltpu.VMEM((rpw,), jnp.float32),
            pltpu.VMEM((rpw,), jnp.int32),
            pltpu.VMEM((rpw,), jnp.int32),
            pltpu.VMEM((rpw,), jnp.int32),
            pltpu.VMEM((rpw,), jnp.float32),
            pltpu.VMEM((rpw,), jnp.float32),
            pltpu.VMEM((_WTAB,), jnp.float32),
            pltpu.VMEM((_B * _K,), jnp.float32),
            pltpu.VMEM((_B * _K,), jnp.float32),
            pltpu.VMEM((_K,), jnp.float32),
            pltpu.VMEM((_K,), jnp.float32),
            pltpu.VMEM((_K,), jnp.float32),
            pltpu.VMEM((_K,), jnp.float32),
            pltpu.VMEM((_B,), jnp.float32),
            pltpu.VMEM((rpw,), jnp.float32),
            pltpu.SemaphoreType.DMA,
        ],
    )
    def body(mvoc_h, bidx_h, frat_h, fmgr_h, fwal_h, fdiv_h, wtab_h,
             out_h,
             mvoc_v, bidx_v, frat_v, fmgr_v, fwal_v, fdiv_v, wtab_v,
             cs_v, ct_v, sw_v, tw_v, sd_v, td_v, cb_v,
             out_v, sem):
        wid = lax.axis_index("s") * _NC + lax.axis_index("c")
        base = wid * rpw
        sl_rows = pl.ds(base, rpw)
        wcp = pltpu.async_copy(wtab_h, wtab_v, sem)
        icps = [
            pltpu.async_copy(mvoc_h.at[sl_rows], mvoc_v, sem),
            pltpu.async_copy(bidx_h.at[sl_rows], bidx_v, sem),
            pltpu.async_copy(frat_h.at[sl_rows], frat_v, sem),
            pltpu.async_copy(fmgr_h.at[sl_rows], fmgr_v, sem),
            pltpu.async_copy(fwal_h.at[sl_rows], fwal_v, sem),
            pltpu.async_copy(fdiv_h.at[sl_rows], fdiv_v, sem),
        ]
        wcp.wait()

        lo = pl.ds(0, _L)
        hi = pl.ds(_L, _L)
        kb_lo = wtab_v[pl.ds(_O_KNOTS, _L)]
        kb_hi = wtab_v[pl.ds(_O_KNOTS + _L, _L)]
        s4 = wtab_v[pl.ds(_O_SCAL, _L)]
        s0 = s4[0] + s4[1] + s4[2] + s4[3]
        cb_v[lo] = wtab_v[pl.ds(_O_BADJ, _L)] + s0
        wlo = wtab_v[pl.ds(_O_WBASE, _L)]
        whi = wtab_v[pl.ds(_O_WBASE + _L, _L)]
        sb_lo, sb_hi = _cumsum2(wlo, whi)
        tb_lo, tb_hi = _cumsum2(wlo * kb_lo, whi * kb_hi)
        for off, s_v, t_v in ((_O_WWAL, sw_v, tw_v), (_O_WDIV, sd_v, td_v)):
            a_lo = wtab_v[pl.ds(off, _L)]
            a_hi = wtab_v[pl.ds(off + _L, _L)]
            r_lo, r_hi = _cumsum2(a_lo, a_hi)
            s_v[lo] = r_lo
            s_v[hi] = r_hi
            r_lo, r_hi = _cumsum2(a_lo * kb_lo, a_hi * kb_hi)
            t_v[lo] = r_lo
            t_v[hi] = r_hi

        @plsc.parallel_loop(0, _B * _K, _K, unroll=2)
        def _bucket(row):
            row_lo = pl.ds(row, _L)
            row_hi = pl.ds(row + _L, _L)
            a_lo = wtab_v[row_lo]
            a_hi = wtab_v[row_hi]
            cbb = plsc.load_gather(cb_v, [jnp.full((_L,), row // _K, jnp.int32)])
            r_lo, r_hi = _cumsum2(a_lo, a_hi)
            cs_v[row_lo] = r_lo + sb_lo
            cs_v[row_hi] = r_hi + sb_hi
            r_lo, r_hi = _cumsum2(a_lo * kb_lo, a_hi * kb_hi)
            ct_v[row_lo] = r_lo + tb_lo - cbb
            ct_v[row_hi] = r_hi + tb_hi - cbb

        for c in icps:
            c.wait()
        scale = jnp.float32(_K - 1)

        @plsc.parallel_loop(0, rpw, _L, unroll=4)
        def _chunk(i):
            sl = pl.ds(i, _L)
            x = mvoc_v[sl]
            j = (x * scale).astype(jnp.int32)
            idx = bidx_v[sl] * _K + j
            acc = x * plsc.load_gather(cs_v, [idx]) - plsc.load_gather(ct_v, [idx])
            xw = fwal_v[sl]
            jw = (xw * scale).astype(jnp.int32)
            acc = acc + (xw * plsc.load_gather(sw_v, [jw]) - plsc.load_gather(tw_v, [jw]))
            xd = fdiv_v[sl]
            jd = (xd * scale).astype(jnp.int32)
            acc = acc + (xd * plsc.load_gather(sd_v, [jd]) - plsc.load_gather(td_v, [jd]))
            acc = acc + plsc.load_gather(wtab_v, [frat_v[sl] + _O_ER])
            acc = acc + plsc.load_gather(wtab_v, [fmgr_v[sl] + _O_EM])
            out_v[sl] = acc

        pltpu.sync_copy(out_v, out_h.at[sl_rows])

    return body


def kernel(mvoc, bucket_idx, feat_rating, feat_manager, feat_wal, feat_div,
           knots, W_base, b_base, W_adj, b_adj, emb_rating, emb_manager,
           W_wal, b_wal, W_div, b_div, bias):
    f32 = jnp.float32
    i32 = jnp.int32
    wtab = jnp.concatenate([
        W_adj.astype(f32).reshape(-1),
        emb_manager.astype(f32).reshape(-1),
        emb_rating.astype(f32).reshape(-1),
        knots.astype(f32),
        W_base.astype(f32),
        W_wal.astype(f32),
        W_div.astype(f32),
        b_adj.astype(f32),
        b_base.astype(f32).reshape(1),
        b_wal.astype(f32).reshape(1),
        b_div.astype(f32).reshape(1),
        bias.astype(f32).reshape(1),
        jnp.zeros((_WTAB - _O_SCAL - 4,), f32),
    ])
    out = _sc_call(mvoc.shape[0])(
        mvoc.astype(f32), bucket_idx.astype(i32), feat_rating.astype(i32),
        feat_manager.astype(i32), feat_wal.astype(f32), feat_div.astype(f32),
        wtab)
    return out[:, None]


# speedup vs baseline: 1.0720x; 1.0720x over previous
import functools

import jax
import jax.numpy as jnp
from jax import lax
from jax.experimental import pallas as pl
from jax.experimental.pallas import tpu as pltpu
from jax.experimental.pallas import tpu_sc as plsc

_NC = 1
_NS = 16
_NW = _NC * _NS
_L = 16
_K = 32
_B = 16
_MGR = 512
_RAT = 24

_O_WADJ = 0
_O_EM = _O_WADJ + _B * _K
_O_ER = _O_EM + _MGR
_O_KNOTS = 1048
_O_WBASE = _O_KNOTS + _K
_O_WWAL = _O_WBASE + _K
_O_WDIV = _O_WWAL + _K
_O_BADJ = _O_WDIV + _K
_O_SCAL = _O_BADJ + _B
_WTAB = 1208


def _cumsum2(lo, hi):
    clo = jnp.cumsum(lo)
    return clo, jnp.cumsum(hi) + jnp.sum(lo)


@functools.lru_cache(maxsize=None)
def _sc_call(n):
    rpw = n // _NW
    mesh = plsc.VectorSubcoreMesh(core_axis_name="c", subcore_axis_name="s",
                                  num_cores=_NC)

    @functools.partial(
        pl.kernel,
        mesh=mesh,
        compiler_params=pltpu.CompilerParams(
            needs_layout_passes=False, use_tc_tiling_on_sc=False),
        out_type=jax.ShapeDtypeStruct((n,), jnp.float32),
        scratch_types=[
            pltpu.VMEM((rpw,), jnp.float32),
            pltpu.VMEM((rpw,), jnp.int32),
            pltpu.VMEM((rpw,), jnp.int32),
            pltpu.VMEM((rpw,), jnp.int32),
            pltpu.VMEM((rpw,), jnp.float32),
            pltpu.VMEM((rpw,), jnp.float32),
            pltpu.VMEM((_WTAB,), jnp.float32),
            pltpu.VMEM((_B * _K,), jnp.float32),
            pltpu.VMEM((_B * _K,), jnp.float32),
            pltpu.VMEM((_K,), jnp.float32),
            pltpu.VMEM((_K,), jnp.float32),
            pltpu.VMEM((_K,), jnp.float32),
            pltpu.VMEM((_K,), jnp.float32),
            pltpu.VMEM((_B,), jnp.float32),
            pltpu.VMEM((rpw,), jnp.float32),
            pltpu.SemaphoreType.DMA,
        ],
    )
    def body(mvoc_h, bidx_h, frat_h, fmgr_h, fwal_h, fdiv_h, wtab_h,
             out_h,
             mvoc_v, bidx_v, frat_v, fmgr_v, fwal_v, fdiv_v, wtab_v,
             cs_v, ct_v, sw_v, tw_v, sd_v, td_v, cb_v,
             out_v, sem):
        wid = lax.axis_index("s") * _NC + lax.axis_index("c")
        base = wid * rpw
        sl_rows = pl.ds(base, rpw)
        wcp = pltpu.async_copy(wtab_h, wtab_v, sem)
        icps = [
            pltpu.async_copy(mvoc_h.at[sl_rows], mvoc_v, sem),
            pltpu.async_copy(bidx_h.at[sl_rows], bidx_v, sem),
            pltpu.async_copy(frat_h.at[sl_rows], frat_v, sem),
            pltpu.async_copy(fmgr_h.at[sl_rows], fmgr_v, sem),
            pltpu.async_copy(fwal_h.at[sl_rows], fwal_v, sem),
            pltpu.async_copy(fdiv_h.at[sl_rows], fdiv_v, sem),
        ]
        wcp.wait()

        lo = pl.ds(0, _L)
        hi = pl.ds(_L, _L)
        kb_lo = wtab_v[pl.ds(_O_KNOTS, _L)]
        kb_hi = wtab_v[pl.ds(_O_KNOTS + _L, _L)]
        s4 = wtab_v[pl.ds(_O_SCAL, _L)]
        s0 = s4[0] + s4[1] + s4[2] + s4[3]
        cb_v[lo] = wtab_v[pl.ds(_O_BADJ, _L)] + s0
        wlo = wtab_v[pl.ds(_O_WBASE, _L)]
        whi = wtab_v[pl.ds(_O_WBASE + _L, _L)]
        sb_lo, sb_hi = _cumsum2(wlo, whi)
        tb_lo, tb_hi = _cumsum2(wlo * kb_lo, whi * kb_hi)
        for off, s_v, t_v in ((_O_WWAL, sw_v, tw_v), (_O_WDIV, sd_v, td_v)):
            a_lo = wtab_v[pl.ds(off, _L)]
            a_hi = wtab_v[pl.ds(off + _L, _L)]
            r_lo, r_hi = _cumsum2(a_lo, a_hi)
            s_v[lo] = r_lo
            s_v[hi] = r_hi
            r_lo, r_hi = _cumsum2(a_lo * kb_lo, a_hi * kb_hi)
            t_v[lo] = r_lo
            t_v[hi] = r_hi

        @plsc.parallel_loop(0, _B * _K, _K, unroll=2)
        def _bucket(row):
            row_lo = pl.ds(row, _L)
            row_hi = pl.ds(row + _L, _L)
            a_lo = wtab_v[row_lo]
            a_hi = wtab_v[row_hi]
            cbb = plsc.load_gather(cb_v, [jnp.full((_L,), row // _K, jnp.int32)])
            r_lo, r_hi = _cumsum2(a_lo, a_hi)
            cs_v[row_lo] = r_lo + sb_lo
            cs_v[row_hi] = r_hi + sb_hi
            r_lo, r_hi = _cumsum2(a_lo * kb_lo, a_hi * kb_hi)
            ct_v[row_lo] = r_lo + tb_lo - cbb
            ct_v[row_hi] = r_hi + tb_hi - cbb

        for c in icps:
            c.wait()
        scale = jnp.float32(_K - 1)

        @plsc.parallel_loop(0, rpw, _L, unroll=4)
        def _chunk(i):
            sl = pl.ds(i, _L)
            x = mvoc_v[sl]
            j = (x * scale).astype(jnp.int32)
            idx = bidx_v[sl] * _K + j
            acc = x * plsc.load_gather(cs_v, [idx]) - plsc.load_gather(ct_v, [idx])
            xw = fwal_v[sl]
            jw = (xw * scale).astype(jnp.int32)
            acc = acc + (xw * plsc.load_gather(sw_v, [jw]) - plsc.load_gather(tw_v, [jw]))
            xd = fdiv_v[sl]
            jd = (xd * scale).astype(jnp.int32)
            acc = acc + (xd * plsc.load_gather(sd_v, [jd]) - plsc.load_gather(td_v, [jd]))
            acc = acc + plsc.load_gather(wtab_v, [frat_v[sl] + _O_ER])
            acc = acc + plsc.load_gather(wtab_v, [fmgr_v[sl] + _O_EM])
            out_v[sl] = acc

        pltpu.sync_copy(out_v, out_h.at[sl_rows])

    return body


def kernel(mvoc, bucket_idx, feat_rating, feat_manager, feat_wal, feat_div,
           knots, W_base, b_base, W_adj, b_adj, emb_rating, emb_manager,
           W_wal, b_wal, W_div, b_div, bias):
    f32 = jnp.float32
    i32 = jnp.int32
    wtab = jnp.concatenate([
        W_adj.astype(f32).reshape(-1),
        emb_manager.astype(f32).reshape(-1),
        emb_rating.astype(f32).reshape(-1),
        knots.astype(f32),
        W_base.astype(f32),
        W_wal.astype(f32),
        W_div.astype(f32),
        b_adj.astype(f32),
        b_base.astype(f32).reshape(1),
        b_wal.astype(f32).reshape(1),
        b_div.astype(f32).reshape(1),
        bias.astype(f32).reshape(1),
        jnp.zeros((_WTAB - _O_SCAL - 4,), f32),
    ])
    out = _sc_call(mvoc.shape[0])(
        mvoc.astype(f32), bucket_idx.astype(i32), feat_rating.astype(i32),
        feat_manager.astype(i32), feat_wal.astype(f32), feat_div.astype(f32),
        wtab)
    return out[:, None]
